# baseline (speedup 1.0000x reference)
import functools

import jax
import jax.numpy as jnp
from jax import lax
from jax.experimental import pallas as pl
from jax.experimental.pallas import tpu as pltpu
from jax.experimental.pallas import tpu_sc as plsc

N_NODES = 10000
N_EDGES = 320000
F = 128

NC = 2
NS = 16
NW = NC * NS

R_MAIN = 632
R_LAST = N_NODES - (NS - 1) * R_MAIN
K = 128
N_CHUNKS = N_EDGES // K
FH = F // NC
CH_TILE = 160
G = 16
R = 4
Z = 64


def _sc_aggregate_body(x_hbm, e_hbm, out_hbm, acc_sh, x_sh, src_v, dst_v,
                       b0, b1, b2, b3, d0, d1, d2, d3,
                       g0, g1, g2, g3, s0, s1, s2, s3):
    bufs = [b0, b1, b2, b3]
    dstks = [d0, d1, d2, d3]
    gsems = [g0, g1, g2, g3]
    ssems = [s0, s1, s2, s3]

    c = lax.axis_index("c")
    s = lax.axis_index("s")

    zero16 = jnp.zeros((16,), jnp.float32)

    def _zrow(i, carry):
        for k in range(FH // 16):
            bufs[0][i, pl.ds(k * 16, 16)] = zero16
        return carry

    lax.fori_loop(0, Z, _zrow, 0)

    base = s * R_MAIN

    def _init_acc(nrows):
        pltpu.sync_copy(x_hbm.at[c, pl.ds(base, nrows)],
                        x_sh.at[pl.ds(base, nrows)])
        nfull = nrows // Z
        rem = nrows - nfull * Z

        def _zacc(i, carry):
            pltpu.sync_copy(bufs[0].at[pl.ds(0, Z)],
                            acc_sh.at[pl.ds(base + i * Z, Z)])
            return carry

        lax.fori_loop(0, nfull, _zacc, 0)
        if rem:
            pltpu.sync_copy(bufs[0].at[pl.ds(0, rem)],
                            acc_sh.at[pl.ds(base + nfull * Z, rem)])

    @pl.when(s < NS - 1)
    def _():
        _init_acc(R_MAIN)

    @pl.when(s == NS - 1)
    def _():
        _init_acc(R_LAST)

    plsc.subcore_barrier()

    cbase = s * CH_TILE

    def _gref(u):
        return x_sh.at[src_v.at[u]]

    def _run_group(goff, n_ch):
        pltpu.sync_copy(e_hbm.at[0, pl.ds(cbase + goff, n_ch)],
                        src_v.at[pl.ds(0, n_ch)])
        pltpu.sync_copy(e_hbm.at[1, pl.ds(cbase + goff, n_ch)],
                        dst_v.at[pl.ds(0, n_ch)])
        for t in range(n_ch + 2):
            if t < n_ch:
                u, b = t, t % R
                if u >= R:
                    pltpu.make_async_copy(
                        bufs[b], acc_sh.at[dstks[b]], ssems[b]).wait()
                pltpu.async_copy(_gref(u), bufs[b], gsems[b])
            if t >= 2:
                u, b = t - 2, (t - 2) % R
                pltpu.make_async_copy(_gref(u), bufs[b], gsems[b]).wait()
                for i in range(K // 16):
                    dstks[b][pl.ds(i * 16, 16)] = (
                        dst_v[u, pl.ds(i * 16, 16)])
                pltpu.async_copy(bufs[b], acc_sh.at[dstks[b]], ssems[b],
                                 add=True)
        for u in range(max(0, n_ch - R), n_ch):
            b = u % R
            pltpu.make_async_copy(bufs[b], acc_sh.at[dstks[b]],
                                  ssems[b]).wait()

    @pl.when(s < NS - 1)
    def _():
        @pl.loop(0, CH_TILE // G)
        def _(g):
            _run_group(g * G, G)

    @pl.when(s == NS - 1)
    def _():
        @pl.loop(0, 6)
        def _(g):
            _run_group(g * G, G)

        _run_group(6 * G, 4)

    plsc.subcore_barrier()

    @pl.when(s < NS - 1)
    def _():
        pltpu.sync_copy(acc_sh.at[pl.ds(base, R_MAIN)],
                        out_hbm.at[c, pl.ds(base, R_MAIN)])

    @pl.when(s == NS - 1)
    def _():
        pltpu.sync_copy(acc_sh.at[pl.ds(base, R_LAST)],
                        out_hbm.at[c, pl.ds(base, R_LAST)])


@jax.jit
def _sc_aggregate(x_pair, e3):
    mesh = plsc.VectorSubcoreMesh(core_axis_name="c", subcore_axis_name="s")
    return pl.kernel(
        _sc_aggregate_body,
        out_type=jax.ShapeDtypeStruct((NC, N_NODES, FH), jnp.float32),
        mesh=mesh,
        compiler_params=pltpu.CompilerParams(use_tc_tiling_on_sc=False),
        scratch_types=[
            pltpu.VMEM_SHARED((N_NODES, FH), jnp.float32),
            pltpu.VMEM_SHARED((N_NODES, FH), jnp.float32),
            pltpu.VMEM((G, K), jnp.int32),
            pltpu.VMEM((G, K), jnp.int32),
        ] + [pltpu.VMEM((K, FH), jnp.float32)] * R
          + [pltpu.VMEM((K,), jnp.int32)] * R
          + [pltpu.SemaphoreType.DMA] * (2 * R),
    )(x_pair, e3)


def _tc_finish_body(acc_ref, w_ref, b_ref, o_ref):
    a = jnp.concatenate([acc_ref[0], acc_ref[1]], axis=1)
    y = jnp.dot(a, w_ref[...], preferred_element_type=jnp.float32)
    o_ref[...] = jnp.tanh(y + b_ref[...])


@jax.jit
def _tc_finish(agg, weight, bias):
    blk = 2000
    return pl.pallas_call(
        _tc_finish_body,
        grid=(N_NODES // blk,),
        in_specs=[
            pl.BlockSpec((NC, blk, FH), lambda i: (0, i, 0)),
            pl.BlockSpec((F, F), lambda i: (0, 0)),
            pl.BlockSpec((1, F), lambda i: (0, 0)),
        ],
        out_specs=pl.BlockSpec((blk, F), lambda i: (i, 0)),
        out_shape=jax.ShapeDtypeStruct((N_NODES, F), jnp.float32),
    )(agg, weight, bias.reshape(1, F))


def kernel(inputs, edge_index, weight, bias):
    x_pair = inputs.reshape(N_NODES, NC, FH).swapaxes(0, 1)
    e3 = edge_index.astype(jnp.int32).reshape(2, N_CHUNKS, K)
    agg = _sc_aggregate(x_pair, e3)
    return _tc_finish(agg, weight, bias)


# speedup vs baseline: 1.3471x; 1.3471x over previous
import functools

import jax
import jax.numpy as jnp
from jax import lax
from jax.experimental import pallas as pl
from jax.experimental.pallas import tpu as pltpu
from jax.experimental.pallas import tpu_sc as plsc

N_NODES = 10000
N_EDGES = 320000
F = 128

NC = 2
NS = 16
NW = NC * NS

R_MAIN = 632
R_LAST = N_NODES - (NS - 1) * R_MAIN
K = 128
N_CHUNKS = N_EDGES // K
CH_TILE = 80
SC0_CHUNKS = 1248
G = 16
KS = 64
R = 5
Z = 64


def _sc_aggregate_body(x_hbm, e_hbm, out_hbm, acc_sh, src_v, dst_v,
                       b0, b1, b2, b3, b4, d0, d1, d2, d3, d4,
                       g0, g1, g2, g3, g4, s0, s1, s2, s3, s4):
    bufs = [b0, b1, b2, b3, b4]
    dstks = [d0, d1, d2, d3, d4]
    gsems = [g0, g1, g2, g3, g4]
    ssems = [s0, s1, s2, s3, s4]

    c = lax.axis_index("c")
    s = lax.axis_index("s")

    zero16 = jnp.zeros((16,), jnp.float32)

    def _zrow(i, carry):
        for k in range(F // 16):
            bufs[0][i, pl.ds(k * 16, 16)] = zero16
        return carry

    lax.fori_loop(0, Z, _zrow, 0)

    base = s * R_MAIN

    def _zero_acc(nrows):
        nfull = nrows // Z
        rem = nrows - nfull * Z

        def _zacc(i, carry):
            pltpu.sync_copy(bufs[0].at[pl.ds(0, Z)],
                            acc_sh.at[pl.ds(base + i * Z, Z)])
            return carry

        lax.fori_loop(0, nfull, _zacc, 0)
        if rem:
            pltpu.sync_copy(bufs[0].at[pl.ds(0, rem)],
                            acc_sh.at[pl.ds(base + nfull * Z, rem)])

    @pl.when(s < NS - 1)
    def _():
        _zero_acc(R_MAIN)

    @pl.when(s == NS - 1)
    def _():
        _zero_acc(R_LAST)

    plsc.subcore_barrier()

    cbase = c * SC0_CHUNKS + s * CH_TILE

    def _gref(u):
        j, h = divmod(u, 2)
        return x_hbm.at[src_v.at[j, pl.ds(h * KS, KS)]]

    def _run_group(goff, n_ch):
        pltpu.sync_copy(e_hbm.at[0, pl.ds(cbase + goff, n_ch)],
                        src_v.at[pl.ds(0, n_ch)])
        pltpu.sync_copy(e_hbm.at[1, pl.ds(cbase + goff, n_ch)],
                        dst_v.at[pl.ds(0, n_ch)])
        nsub = n_ch * 2
        for t in range(nsub + 2):
            if t < nsub:
                u, b = t, t % R
                if u >= R:
                    pltpu.make_async_copy(
                        bufs[b], acc_sh.at[dstks[b]], ssems[b]).wait()
                pltpu.async_copy(_gref(u), bufs[b], gsems[b])
            if t >= 2:
                u, b = t - 2, (t - 2) % R
                pltpu.make_async_copy(_gref(u), bufs[b], gsems[b]).wait()
                j, h = divmod(u, 2)
                for i in range(KS // 16):
                    dstks[b][pl.ds(i * 16, 16)] = (
                        dst_v[j, pl.ds(h * KS + i * 16, 16)])
                pltpu.async_copy(bufs[b], acc_sh.at[dstks[b]], ssems[b],
                                 add=True)
        for u in range(max(0, nsub - R), nsub):
            b = u % R
            pltpu.make_async_copy(bufs[b], acc_sh.at[dstks[b]],
                                  ssems[b]).wait()

    @pl.when(s < NS - 1)
    def _():
        @pl.loop(0, CH_TILE // G)
        def _(g):
            _run_group(g * G, G)

    @pl.when(s == NS - 1)
    def _():
        @pl.loop(0, 3)
        def _(g):
            _run_group(g * G, G)

        @pl.when(c == 1)
        def _():
            _run_group(3 * G, 4)

    plsc.subcore_barrier()

    @pl.when(s < NS - 1)
    def _():
        pltpu.sync_copy(acc_sh.at[pl.ds(base, R_MAIN)],
                        out_hbm.at[c, pl.ds(base, R_MAIN)])

    @pl.when(s == NS - 1)
    def _():
        pltpu.sync_copy(acc_sh.at[pl.ds(base, R_LAST)],
                        out_hbm.at[c, pl.ds(base, R_LAST)])


@jax.jit
def _sc_aggregate(x, e3):
    mesh = plsc.VectorSubcoreMesh(core_axis_name="c", subcore_axis_name="s")
    return pl.kernel(
        _sc_aggregate_body,
        out_type=jax.ShapeDtypeStruct((NC, N_NODES, F), jnp.float32),
        mesh=mesh,
        scratch_types=[
            pltpu.VMEM_SHARED((N_NODES, F), jnp.float32),
            pltpu.VMEM((G, K), jnp.int32),
            pltpu.VMEM((G, K), jnp.int32),
        ] + [pltpu.VMEM((KS, F), jnp.float32)] * R
          + [pltpu.VMEM((KS,), jnp.int32)] * R
          + [pltpu.SemaphoreType.DMA] * (2 * R),
    )(x, e3)


def _tc_finish_body(acc_ref, w_ref, b_ref, o_ref):
    a = acc_ref[0] + acc_ref[1]
    y = jnp.dot(a, w_ref[...], preferred_element_type=jnp.float32)
    o_ref[...] = jnp.tanh(y + b_ref[...])


@jax.jit
def _tc_finish(agg, weight, bias):
    blk = 2000
    return pl.pallas_call(
        _tc_finish_body,
        grid=(N_NODES // blk,),
        in_specs=[
            pl.BlockSpec((NC, blk, F), lambda i: (0, i, 0)),
            pl.BlockSpec((F, F), lambda i: (0, 0)),
            pl.BlockSpec((1, F), lambda i: (0, 0)),
        ],
        out_specs=pl.BlockSpec((blk, F), lambda i: (i, 0)),
        out_shape=jax.ShapeDtypeStruct((N_NODES, F), jnp.float32),
    )(agg, weight, bias.reshape(1, F))


def kernel(inputs, edge_index, weight, bias):
    e3 = edge_index.astype(jnp.int32).reshape(2, N_CHUNKS, K)
    agg = _sc_aggregate(inputs, e3)
    return _tc_finish(agg, weight, bias)
